# Initial kernel scaffold; baseline (speedup 1.0000x reference)
import jax
import jax.numpy as jnp
from jax.experimental import pallas as pl


def kernel(x, edge_index, W_l, W_r, b):
    raise NotImplementedError("write your pallas kernel here")


# speedup vs baseline: 4.7455x; 4.7455x over previous
import functools

import jax
import jax.numpy as jnp
from jax import lax
from jax.experimental import pallas as pl
from jax.experimental.pallas import tpu as pltpu
from jax.experimental.pallas import tpu_sc as plsc

N = 10000
E = 320000
D = 128
OUT = 128

NC = 2
NS = 16
NW = NC * NS
EPW = E // NW
CHUNK = 80
NCHUNK = EPW // CHUNK
N_PAD = 10112
DEGW = 16


def _sc_values(x, src, dst, z_acc):
    mesh = plsc.VectorSubcoreMesh(core_axis_name="c", subcore_axis_name="s")

    @functools.partial(
        pl.kernel,
        mesh=mesh,
        out_type=jax.ShapeDtypeStruct((NC, N_PAD, D), jnp.float32),
        scratch_types=[
            pltpu.VMEM((CHUNK,), jnp.int32),
            pltpu.VMEM((CHUNK,), jnp.int32),
            pltpu.VMEM((CHUNK, D), jnp.float32),
            pltpu.VMEM_SHARED((N_PAD, D), jnp.float32),
            pltpu.SemaphoreType.DMA,
        ],
    )
    def agg(x_hbm, src_hbm, dst_hbm, zacc_hbm, part_hbm,
            src_v, dst_v, rows_v, acc_sh, sem):
        c = lax.axis_index("c")
        s = lax.axis_index("s")
        wid = c * NS + s

        @pl.when(s == 0)
        def _():
            pltpu.sync_copy(zacc_hbm, acc_sh)
        plsc.subcore_barrier()

        def step(i, carry):
            off = wid * EPW + i * CHUNK
            pltpu.sync_copy(src_hbm.at[pl.ds(off, CHUNK)], src_v)
            pltpu.sync_copy(dst_hbm.at[pl.ds(off, CHUNK)], dst_v)
            pltpu.async_copy(x_hbm.at[src_v], rows_v, sem).wait()
            pltpu.sync_copy(rows_v, acc_sh.at[dst_v], add=True)
            return carry
        lax.fori_loop(0, NCHUNK, step, 0)
        plsc.subcore_barrier()

        @pl.when(s == 0)
        def _():
            pltpu.sync_copy(acc_sh, part_hbm.at[c])

    return agg(x, src, dst, z_acc)


def _sc_degree(dst, z_deg, ones_h):
    mesh = plsc.VectorSubcoreMesh(core_axis_name="c", subcore_axis_name="s")

    @functools.partial(
        pl.kernel,
        mesh=mesh,
        out_type=jax.ShapeDtypeStruct((NC, N_PAD, D), jnp.float32),
        scratch_types=[
            pltpu.VMEM((CHUNK,), jnp.int32),
            pltpu.VMEM((CHUNK, D), jnp.float32),
            pltpu.VMEM_SHARED((N_PAD, D), jnp.float32),
        ],
    )
    def deg(dst_hbm, zdeg_hbm, ones_hbm, degp_hbm,
            dst_v, ones_v, deg_sh):
        c = lax.axis_index("c")
        s = lax.axis_index("s")
        wid = c * NS + s

        pltpu.sync_copy(ones_hbm, ones_v)

        @pl.when(s == 0)
        def _():
            pltpu.sync_copy(zdeg_hbm, deg_sh)
        plsc.subcore_barrier()

        def step(i, carry):
            off = wid * EPW + i * CHUNK
            pltpu.sync_copy(dst_hbm.at[pl.ds(off, CHUNK)], dst_v)
            pltpu.sync_copy(ones_v, deg_sh.at[dst_v], add=True)
            return carry
        lax.fori_loop(0, NCHUNK, step, 0)
        plsc.subcore_barrier()

        @pl.when(s == 0)
        def _():
            pltpu.sync_copy(deg_sh, degp_hbm.at[c])

    return deg(dst, z_deg, ones_h)


TB = 1000


def _tc_combine(part, degp, x, W_l, W_r, b2):
    def body(p_ref, d_ref, x_ref, wl_ref, wr_ref, b_ref, o_ref):
        summed = p_ref[0] + p_ref[1]
        deg = d_ref[0, :, 0:1] + d_ref[1, :, 0:1]
        mean = summed / jnp.maximum(deg, 1.0)
        o_ref[...] = (
            jnp.dot(mean, wl_ref[...], preferred_element_type=jnp.float32)
            + jnp.dot(x_ref[...], wr_ref[...], preferred_element_type=jnp.float32)
            + b_ref[...]
        )

    return pl.pallas_call(
        body,
        grid=(N // TB,),
        in_specs=[
            pl.BlockSpec((NC, TB, D), lambda i: (0, i, 0)),
            pl.BlockSpec((NC, TB, D), lambda i: (0, i, 0)),
            pl.BlockSpec((TB, D), lambda i: (i, 0)),
            pl.BlockSpec((D, OUT), lambda i: (0, 0)),
            pl.BlockSpec((D, OUT), lambda i: (0, 0)),
            pl.BlockSpec((1, OUT), lambda i: (0, 0)),
        ],
        out_specs=pl.BlockSpec((TB, OUT), lambda i: (i, 0)),
        out_shape=jax.ShapeDtypeStruct((N, OUT), jnp.float32),
    )(part, degp, x, W_l, W_r, b2)


@jax.jit
def _run(x, edge_index, W_l, W_r, b):
    src = edge_index[0].astype(jnp.int32)
    dst = edge_index[1].astype(jnp.int32)
    z_acc = jnp.zeros((N_PAD, D), jnp.float32)
    z_deg = jnp.zeros((N_PAD, D), jnp.float32)
    ones_h = jnp.ones((CHUNK, D), jnp.float32)
    part = _sc_values(x, src, dst, z_acc)
    degp = _sc_degree(dst, z_deg, ones_h)
    return _tc_combine(part, degp, x, W_l, W_r, b.reshape(1, OUT))


def kernel(x, edge_index, W_l, W_r, b):
    return _run(x, edge_index, W_l, W_r, b)


# speedup vs baseline: 7.4618x; 1.5724x over previous
import functools

import jax
import jax.numpy as jnp
from jax import lax
from jax.experimental import pallas as pl
from jax.experimental.pallas import tpu as pltpu
from jax.experimental.pallas import tpu_sc as plsc

N = 10000
E = 320000
D = 128
OUT = 128

NC = 2
NS = 16
NW = NC * NS
EPW = E // NW
CHUNK = 80
NCHUNK = EPW // CHUNK
NPAIR = (NCHUNK - 1) // 2
N_PAD = 10112


def _sc_values(x, src, dst, z_acc):
    mesh = plsc.VectorSubcoreMesh(core_axis_name="c", subcore_axis_name="s")

    @functools.partial(
        pl.kernel,
        mesh=mesh,
        out_type=jax.ShapeDtypeStruct((NC, N_PAD, D), jnp.float32),
        scratch_types=[
            pltpu.VMEM((CHUNK,), jnp.int32),
            pltpu.VMEM((CHUNK,), jnp.int32),
            pltpu.VMEM((CHUNK, D), jnp.float32),
            pltpu.VMEM((CHUNK,), jnp.int32),
            pltpu.VMEM((CHUNK,), jnp.int32),
            pltpu.VMEM((CHUNK, D), jnp.float32),
            pltpu.VMEM_SHARED((N_PAD, D), jnp.float32),
            pltpu.SemaphoreType.DMA,
            pltpu.SemaphoreType.DMA,
        ],
    )
    def agg(x_hbm, src_hbm, dst_hbm, zacc_hbm, part_hbm,
            src_v0, dst_v0, rows_v0, src_v1, dst_v1, rows_v1,
            acc_sh, gsem0, gsem1):
        c = lax.axis_index("c")
        s = lax.axis_index("s")
        wid = c * NS + s
        base = wid * EPW

        @pl.when(s == 0)
        def _():
            pltpu.sync_copy(zacc_hbm, acc_sh)
        plsc.subcore_barrier()

        def load_idx(j, sv, dv):
            off = base + j * CHUNK
            pltpu.sync_copy(src_hbm.at[pl.ds(off, CHUNK)], sv)
            pltpu.sync_copy(dst_hbm.at[pl.ds(off, CHUNK)], dv)

        load_idx(0, src_v0, dst_v0)
        load_idx(1, src_v1, dst_v1)
        pltpu.async_copy(x_hbm.at[src_v0], rows_v0, gsem0)

        def pair(k, carry):
            j = 2 * k
            pltpu.make_async_copy(x_hbm.at[src_v0], rows_v0, gsem0).wait()
            pltpu.async_copy(x_hbm.at[src_v1], rows_v1, gsem1)
            pltpu.sync_copy(rows_v0, acc_sh.at[dst_v0], add=True)
            load_idx(j + 2, src_v0, dst_v0)
            pltpu.make_async_copy(x_hbm.at[src_v1], rows_v1, gsem1).wait()
            pltpu.async_copy(x_hbm.at[src_v0], rows_v0, gsem0)
            pltpu.sync_copy(rows_v1, acc_sh.at[dst_v1], add=True)
            jn = jnp.minimum(j + 3, NCHUNK - 1)
            load_idx(jn, src_v1, dst_v1)
            return carry
        lax.fori_loop(0, NPAIR, pair, 0)

        pltpu.make_async_copy(x_hbm.at[src_v0], rows_v0, gsem0).wait()
        pltpu.sync_copy(rows_v0, acc_sh.at[dst_v0], add=True)
        plsc.subcore_barrier()

        @pl.when(s == 0)
        def _():
            pltpu.sync_copy(acc_sh, part_hbm.at[c])

    return agg(x, src, dst, z_acc)


def _sc_degree(dst, z_deg, ones_h):
    mesh = plsc.VectorSubcoreMesh(core_axis_name="c", subcore_axis_name="s")

    @functools.partial(
        pl.kernel,
        mesh=mesh,
        out_type=jax.ShapeDtypeStruct((NC, N_PAD, D), jnp.float32),
        scratch_types=[
            pltpu.VMEM((CHUNK,), jnp.int32),
            pltpu.VMEM((CHUNK,), jnp.int32),
            pltpu.VMEM((CHUNK, D), jnp.float32),
            pltpu.VMEM_SHARED((N_PAD, D), jnp.float32),
            pltpu.SemaphoreType.DMA,
            pltpu.SemaphoreType.DMA,
        ],
    )
    def deg(dst_hbm, zdeg_hbm, ones_hbm, degp_hbm,
            dst_v0, dst_v1, ones_v, deg_sh, isem0, isem1):
        c = lax.axis_index("c")
        s = lax.axis_index("s")
        wid = c * NS + s
        base = wid * EPW

        pltpu.sync_copy(ones_hbm, ones_v)

        @pl.when(s == 0)
        def _():
            pltpu.sync_copy(zdeg_hbm, deg_sh)
        plsc.subcore_barrier()

        pltpu.sync_copy(dst_hbm.at[pl.ds(base, CHUNK)], dst_v0)

        def pair(k, carry):
            j = 2 * k
            pltpu.async_copy(dst_hbm.at[pl.ds(base + (j + 1) * CHUNK, CHUNK)],
                             dst_v1, isem1)
            pltpu.sync_copy(ones_v, deg_sh.at[dst_v0], add=True)
            pltpu.make_async_copy(dst_hbm.at[pl.ds(0, CHUNK)], dst_v1,
                                  isem1).wait()
            jn = jnp.minimum(j + 2, NCHUNK - 1)
            pltpu.async_copy(dst_hbm.at[pl.ds(base + jn * CHUNK, CHUNK)],
                             dst_v0, isem0)
            pltpu.sync_copy(ones_v, deg_sh.at[dst_v1], add=True)
            pltpu.make_async_copy(dst_hbm.at[pl.ds(0, CHUNK)], dst_v0,
                                  isem0).wait()
            return carry
        lax.fori_loop(0, NPAIR, pair, 0)

        pltpu.sync_copy(ones_v, deg_sh.at[dst_v0], add=True)
        plsc.subcore_barrier()

        @pl.when(s == 0)
        def _():
            pltpu.sync_copy(deg_sh, degp_hbm.at[c])

    return deg(dst, z_deg, ones_h)


TB = 1000


def _tc_combine(part, degp, x, W_l, W_r, b2):
    def body(p_ref, d_ref, x_ref, wl_ref, wr_ref, b_ref, o_ref):
        summed = p_ref[0] + p_ref[1]
        deg = d_ref[0, :, 0:1] + d_ref[1, :, 0:1]
        mean = summed / jnp.maximum(deg, 1.0)
        o_ref[...] = (
            jnp.dot(mean, wl_ref[...], preferred_element_type=jnp.float32)
            + jnp.dot(x_ref[...], wr_ref[...], preferred_element_type=jnp.float32)
            + b_ref[...]
        )

    return pl.pallas_call(
        body,
        grid=(N // TB,),
        in_specs=[
            pl.BlockSpec((NC, TB, D), lambda i: (0, i, 0)),
            pl.BlockSpec((NC, TB, D), lambda i: (0, i, 0)),
            pl.BlockSpec((TB, D), lambda i: (i, 0)),
            pl.BlockSpec((D, OUT), lambda i: (0, 0)),
            pl.BlockSpec((D, OUT), lambda i: (0, 0)),
            pl.BlockSpec((1, OUT), lambda i: (0, 0)),
        ],
        out_specs=pl.BlockSpec((TB, OUT), lambda i: (i, 0)),
        out_shape=jax.ShapeDtypeStruct((N, OUT), jnp.float32),
    )(part, degp, x, W_l, W_r, b2)


@jax.jit
def _run(x, edge_index, W_l, W_r, b):
    src = edge_index[0].astype(jnp.int32)
    dst = edge_index[1].astype(jnp.int32)
    z_acc = jnp.zeros((N_PAD, D), jnp.float32)
    z_deg = jnp.zeros((N_PAD, D), jnp.float32)
    ones_h = jnp.ones((CHUNK, D), jnp.float32)
    part = _sc_values(x, src, dst, z_acc)
    degp = _sc_degree(dst, z_deg, ones_h)
    return _tc_combine(part, degp, x, W_l, W_r, b.reshape(1, OUT))


def kernel(x, edge_index, W_l, W_r, b):
    return _run(x, edge_index, W_l, W_r, b)


# speedup vs baseline: 8.8619x; 1.1876x over previous
import functools

import jax
import jax.numpy as jnp
from jax import lax
from jax.experimental import pallas as pl
from jax.experimental.pallas import tpu as pltpu
from jax.experimental.pallas import tpu_sc as plsc

N = 10000
E = 320000
D = 128
OUT = 128

NC = 2
NS = 16
NW = NC * NS
EPW = E // NW
CHUNK = 80
NCHUNK = EPW // CHUNK
NPAIR = (NCHUNK - 1) // 2
N_PAD = 10112


def _sc_values(x, src, dst, z_acc):
    mesh = plsc.VectorSubcoreMesh(core_axis_name="c", subcore_axis_name="s")

    @functools.partial(
        pl.kernel,
        mesh=mesh,
        out_type=jax.ShapeDtypeStruct((NC, N_PAD, D), jnp.float32),
        scratch_types=(
            [pltpu.VMEM((CHUNK,), jnp.int32) for _ in range(4)]
            + [pltpu.VMEM((CHUNK,), jnp.int32) for _ in range(4)]
            + [pltpu.VMEM((CHUNK, D), jnp.float32) for _ in range(4)]
            + [pltpu.VMEM_SHARED((N_PAD, D), jnp.float32)]
            + [pltpu.SemaphoreType.DMA for _ in range(8)]
        ),
    )
    def agg(x_hbm, src_hbm, dst_hbm, zacc_hbm, part_hbm, *refs):
        src_v = refs[0:4]
        dst_v = refs[4:8]
        rows_v = refs[8:12]
        acc_sh = refs[12]
        gsem = refs[13:17]
        isem = refs[17:21]
        c = lax.axis_index("c")
        s = lax.axis_index("s")
        wid = c * NS + s
        base = wid * EPW

        @pl.when(s == 0)
        def _():
            pltpu.sync_copy(zacc_hbm, acc_sh)
        plsc.subcore_barrier()

        def load_idx(j, m):
            off = base + j * CHUNK
            pltpu.async_copy(src_hbm.at[pl.ds(off, CHUNK)], src_v[m], isem[m])
            pltpu.async_copy(dst_hbm.at[pl.ds(off, CHUNK)], dst_v[m], isem[m])

        def wait_idx(m):
            pltpu.make_async_copy(src_hbm.at[pl.ds(0, CHUNK)], src_v[m],
                                  isem[m]).wait()
            pltpu.make_async_copy(dst_hbm.at[pl.ds(0, CHUNK)], dst_v[m],
                                  isem[m]).wait()

        def wait_gather(m):
            pltpu.make_async_copy(x_hbm.at[src_v[m]], rows_v[m],
                                  gsem[m]).wait()

        for m in range(4):
            load_idx(m, m)
        for m in range(3):
            wait_idx(m)
            pltpu.async_copy(x_hbm.at[src_v[m]], rows_v[m], gsem[m])

        def quad(k, carry):
            j4 = 4 * k
            for i in range(4):
                jj = j4 + i
                m = (i + 3) % 4
                wait_gather(i)
                wait_idx(m)
                pltpu.async_copy(x_hbm.at[src_v[m]], rows_v[m], gsem[m])
                pltpu.sync_copy(rows_v[i], acc_sh.at[dst_v[i]], add=True)
                load_idx(jnp.minimum(jj + 4, NCHUNK - 1), i)
            return carry
        lax.fori_loop(0, (NCHUNK - 1) // 4, quad, 0)

        wait_gather(0)
        pltpu.sync_copy(rows_v[0], acc_sh.at[dst_v[0]], add=True)
        wait_gather(1)
        wait_gather(2)
        wait_idx(3)
        plsc.subcore_barrier()

        @pl.when(s == 0)
        def _():
            pltpu.sync_copy(acc_sh, part_hbm.at[c])

    return agg(x, src, dst, z_acc)


def _sc_degree(dst, z_deg, ones_h):
    mesh = plsc.VectorSubcoreMesh(core_axis_name="c", subcore_axis_name="s")

    @functools.partial(
        pl.kernel,
        mesh=mesh,
        out_type=jax.ShapeDtypeStruct((NC, N_PAD, D), jnp.float32),
        scratch_types=[
            pltpu.VMEM((CHUNK,), jnp.int32),
            pltpu.VMEM((CHUNK,), jnp.int32),
            pltpu.VMEM((CHUNK, D), jnp.float32),
            pltpu.VMEM_SHARED((N_PAD, D), jnp.float32),
            pltpu.SemaphoreType.DMA,
            pltpu.SemaphoreType.DMA,
        ],
    )
    def deg(dst_hbm, zdeg_hbm, ones_hbm, degp_hbm,
            dst_v0, dst_v1, ones_v, deg_sh, isem0, isem1):
        c = lax.axis_index("c")
        s = lax.axis_index("s")
        wid = c * NS + s
        base = wid * EPW

        pltpu.sync_copy(ones_hbm, ones_v)

        @pl.when(s == 0)
        def _():
            pltpu.sync_copy(zdeg_hbm, deg_sh)
        plsc.subcore_barrier()

        pltpu.sync_copy(dst_hbm.at[pl.ds(base, CHUNK)], dst_v0)

        def pair(k, carry):
            j = 2 * k
            pltpu.async_copy(dst_hbm.at[pl.ds(base + (j + 1) * CHUNK, CHUNK)],
                             dst_v1, isem1)
            pltpu.sync_copy(ones_v, deg_sh.at[dst_v0], add=True)
            pltpu.make_async_copy(dst_hbm.at[pl.ds(0, CHUNK)], dst_v1,
                                  isem1).wait()
            jn = jnp.minimum(j + 2, NCHUNK - 1)
            pltpu.async_copy(dst_hbm.at[pl.ds(base + jn * CHUNK, CHUNK)],
                             dst_v0, isem0)
            pltpu.sync_copy(ones_v, deg_sh.at[dst_v1], add=True)
            pltpu.make_async_copy(dst_hbm.at[pl.ds(0, CHUNK)], dst_v0,
                                  isem0).wait()
            return carry
        lax.fori_loop(0, NPAIR, pair, 0)

        pltpu.sync_copy(ones_v, deg_sh.at[dst_v0], add=True)
        plsc.subcore_barrier()

        @pl.when(s == 0)
        def _():
            pltpu.sync_copy(deg_sh, degp_hbm.at[c])

    return deg(dst, z_deg, ones_h)


TB = 1000


def _tc_combine(part, degp, x, W_l, W_r, b2):
    def body(p_ref, d_ref, x_ref, wl_ref, wr_ref, b_ref, o_ref):
        summed = p_ref[0] + p_ref[1]
        deg = d_ref[0, :, 0:1] + d_ref[1, :, 0:1]
        mean = summed / jnp.maximum(deg, 1.0)
        o_ref[...] = (
            jnp.dot(mean, wl_ref[...], preferred_element_type=jnp.float32)
            + jnp.dot(x_ref[...], wr_ref[...], preferred_element_type=jnp.float32)
            + b_ref[...]
        )

    return pl.pallas_call(
        body,
        grid=(N // TB,),
        in_specs=[
            pl.BlockSpec((NC, TB, D), lambda i: (0, i, 0)),
            pl.BlockSpec((NC, TB, 8), lambda i: (0, i, 0)),
            pl.BlockSpec((TB, D), lambda i: (i, 0)),
            pl.BlockSpec((D, OUT), lambda i: (0, 0)),
            pl.BlockSpec((D, OUT), lambda i: (0, 0)),
            pl.BlockSpec((1, OUT), lambda i: (0, 0)),
        ],
        out_specs=pl.BlockSpec((TB, OUT), lambda i: (i, 0)),
        out_shape=jax.ShapeDtypeStruct((N, OUT), jnp.float32),
    )(part, degp, x, W_l, W_r, b2)


@jax.jit
def _run(x, edge_index, W_l, W_r, b):
    src = edge_index[0].astype(jnp.int32)
    dst = edge_index[1].astype(jnp.int32)
    z_acc = jnp.zeros((N_PAD, D), jnp.float32)
    ones_h = jnp.ones((CHUNK, D), jnp.float32)
    part = _sc_values(x, src, dst, z_acc)
    degp = _sc_degree(dst, z_acc, ones_h)
    return _tc_combine(part, degp[:, :, :8], x, W_l, W_r, b.reshape(1, OUT))


def kernel(x, edge_index, W_l, W_r, b):
    return _run(x, edge_index, W_l, W_r, b)
